# baseline (speedup 1.0000x reference)
import functools
import numpy as np
import jax
import jax.numpy as jnp
from jax.experimental import pallas as pl
from jax.experimental.pallas import tpu as pltpu


def _fold_bn(gamma, beta, mean, var, eps=1e-5):
    scale = gamma / jnp.sqrt(var + eps)
    shift = beta - mean * scale
    return scale, shift


def _unit_kernel(x_ref, w1t_ref, b1_ref, wdw_ref, w2t_ref, b3_ref,
                 o_ref, *, h, w, ksize, nsub, wpad):
    hw = h * w

    for j in range(nsub):
        x = x_ref[j]
        y = jnp.dot(w1t_ref[...], x,
                    preferred_element_type=jnp.float32)
        y = jnp.maximum(y.astype(jnp.bfloat16) + b1_ref[...], 0)

        c = y.shape[0]
        z1 = jnp.zeros((c, 1), jnp.bfloat16)
        zw = jnp.zeros((c, w), jnp.bfloat16)
        t0 = jnp.concatenate([z1, y[:, :hw - 1]], 1)
        t1 = y
        t2 = jnp.concatenate([y[:, 1:], z1], 1)

        def wk(kh, kw):
            base = (3 * kh + kw) * wpad
            return wdw_ref[:, base:base + hw]

        def urow(kh):
            return t0 * wk(kh, 0) + t1 * wk(kh, 1) + t2 * wk(kh, 2)

        u0, u2 = urow(0), urow(2)
        z = (urow(1) + jnp.concatenate([zw, u0[:, :hw - w]], 1)
             + jnp.concatenate([u2[:, w:], zw], 1))

        out = jnp.dot(w2t_ref[...], z,
                      preferred_element_type=jnp.float32)
        out = jnp.maximum(out.astype(jnp.bfloat16) + b3_ref[...] + x, 0)
        o_ref[j] = out


def kernel(x, w1, wdw, w2,
           bn1_gamma, bn1_beta, bn1_mean, bn1_var,
           bn2_gamma, bn2_beta, bn2_mean, bn2_var,
           bn3_gamma, bn3_beta, bn3_mean, bn3_var):
    inp, oup, group = 256, 256, 4
    mid, ksize = 256, 3
    n, cin, h, w = x.shape
    assert cin == inp and oup == inp
    hw = h * w

    sc1, sh1 = _fold_bn(bn1_gamma, bn1_beta, bn1_mean, bn1_var)
    sc2, sh2 = _fold_bn(bn2_gamma, bn2_beta, bn2_mean, bn2_var)
    sc3, sh3 = _fold_bn(bn3_gamma, bn3_beta, bn3_mean, bn3_var)

    gc = mid // group
    cin_g, oc_g = inp // group, mid // group
    m1 = (np.arange(mid)[:, None] // oc_g == np.arange(inp)[None, :] // cin_g)
    w1t = (jnp.tile(w1[:, :, 0, 0], (1, group)) * m1
           * sc1[:, None]).astype(jnp.bfloat16)
    perm = np.arange(mid).reshape(gc, group).T.reshape(-1)
    iperm = np.argsort(perm)
    oc2 = oup // group
    S = np.zeros((gc, mid), np.float32)
    S[iperm % gc, np.arange(mid)] = 1.0
    M = (np.arange(oup)[:, None] // oc2 == (iperm // gc)[None, :])
    w2sq = w2[:, :, 0, 0]
    w2t_f32 = (w2sq @ S) * M * sc3[:, None]
    w2t = w2t_f32.astype(jnp.bfloat16)
    b1 = sh1[:, None].astype(jnp.bfloat16)
    b3 = (w2t_f32 @ sh2 + sh3)[:, None].astype(jnp.bfloat16)

    wpad = ((hw + 127) // 128) * 128
    colj = np.arange(hw) % w
    kmask = np.ones((ksize, wpad), np.float32)
    kmask[:, hw:] = 0.0
    kmask[0, :hw] = (colj >= 1)
    kmask[2, :hw] = (colj <= w - 2)
    wdw_sc = wdw[:, 0, :, :] * sc2[:, None, None]
    wdw_cm = (wdw_sc[:, :, :, None] * kmask[None, None, :, :]
              ).reshape(mid, ksize * ksize * wpad).astype(jnp.bfloat16)

    xcm = x.reshape(n, inp, hw).astype(jnp.bfloat16)

    nsub = 1
    kern = functools.partial(_unit_kernel, h=h, w=w, ksize=ksize,
                             nsub=nsub, wpad=wpad)
    out = pl.pallas_call(
        kern,
        out_shape=jax.ShapeDtypeStruct((n, oup, hw), jnp.bfloat16),
        grid=(n // nsub,),
        in_specs=[
            pl.BlockSpec((nsub, inp, hw), lambda i: (i, 0, 0)),
            pl.BlockSpec((mid, inp), lambda i: (0, 0)),
            pl.BlockSpec((mid, 1), lambda i: (0, 0)),
            pl.BlockSpec((mid, ksize * ksize * wpad), lambda i: (0, 0)),
            pl.BlockSpec((oup, mid), lambda i: (0, 0)),
            pl.BlockSpec((oup, 1), lambda i: (0, 0)),
        ],
        out_specs=pl.BlockSpec((nsub, oup, hw), lambda i: (i, 0, 0)),
        compiler_params=pltpu.CompilerParams(
            dimension_semantics=("arbitrary",),
            vmem_limit_bytes=int(32 << 20)),
    )(xcm, w1t, b1, wdw_cm, w2t, b3)
    return out.astype(jnp.float32).reshape(n, oup, h, w)
